# baseline (speedup 1.0000x reference)
import functools

import jax
import jax.numpy as jnp
from jax import lax
from jax.experimental import pallas as pl
from jax.experimental.pallas import tpu as pltpu
from jax.experimental.pallas import tpu_sc as plsc

DIM = 64
L = 16

_GATHER_DN = lax.GatherDimensionNumbers(
    offset_dims=(), collapsed_slice_dims=(0,), start_index_map=(0,))


def _shuffle(v, idx):
  return lax.gather(v, idx[:, None], _GATHER_DN, slice_sizes=(1,),
                    mode=lax.GatherScatterMode.PROMISE_IN_BOUNDS)


def _sc_body(x_hbm, y_hbm, at_hbm, bt_hbm, cc_hbm, out_hbm,
             xi_v, col_a, col_b, cx_v, cy_v, out_v, sem):
  cid = lax.axis_index("c")
  sid = lax.axis_index("s")

  @pl.when(jnp.logical_and(cid == 0, sid == 0))
  def _():
    pltpu.sync_copy(x_hbm, xi_v.at[pl.ds(0, 1)])
    pltpu.sync_copy(y_hbm, xi_v.at[pl.ds(8, 1)])
    iv = xi_v[...]
    xs = iv[0]
    ys = iv[8]
    xt = pl.multiple_of((xs // 128) * 128, 128)
    yt = pl.multiple_of((ys // 128) * 128, 128)
    xb = (xs // 8) * 8
    yb = (ys // 8) * 8

    d0 = pltpu.make_async_copy(at_hbm.at[:, pl.ds(xt, 128)], col_a, sem)
    d1 = pltpu.make_async_copy(bt_hbm.at[:, pl.ds(yt, 128)], col_b, sem)
    d2 = pltpu.make_async_copy(cc_hbm.at[pl.ds(xb, 8)], cx_v.at[pl.ds(0, 8)],
                               sem)
    d3 = pltpu.make_async_copy(cc_hbm.at[pl.ds(100000 + yb, 8)],
                               cy_v.at[pl.ds(0, 8)], sem)
    d0.start()
    d1.start()
    d2.start()
    d3.start()
    d0.wait()
    d1.wait()
    d2.wait()
    d3.wait()

    ii = lax.iota(jnp.int32, L)
    xa = jnp.full((L,), xs - xt, jnp.int32)
    ya = jnp.full((L,), ys - yt, jnp.int32)
    acc = jnp.zeros((L,), jnp.float32)
    for i in range(DIM // L):
      av = plsc.load_gather(col_a, [ii + (i * L), xa])
      bv = plsc.load_gather(col_b, [ii + (i * L), ya])
      acc = acc + av * bv

    cx = _shuffle(cx_v[...], jnp.full((L,), xs - xb, jnp.int32))
    cy = _shuffle(cy_v[...], jnp.full((L,), ys - yb, jnp.int32))
    zero = jnp.zeros((L,), jnp.float32)
    s = acc + jnp.where(ii == 0, cx + cy, zero)
    for k in (8, 4, 2, 1):
      s = s + _shuffle(s, ii ^ k)
    out_v[...] = s
    pltpu.sync_copy(out_v.at[pl.ds(0, 1)], out_hbm)


_sc_kernel = functools.partial(
    pl.kernel,
    out_type=jax.ShapeDtypeStruct((1,), jnp.float32),
    mesh=plsc.VectorSubcoreMesh(core_axis_name="c", subcore_axis_name="s",
                                num_cores=1),
    scratch_types=[
        pltpu.VMEM((L,), jnp.int32),
        pltpu.VMEM((DIM, 128), jnp.float32),
        pltpu.VMEM((DIM, 128), jnp.float32),
        pltpu.VMEM((L,), jnp.float32),
        pltpu.VMEM((L,), jnp.float32),
        pltpu.VMEM((L,), jnp.float32),
        pltpu.SemaphoreType.DMA,
    ],
    compiler_params=pltpu.CompilerParams(
        use_tc_tiling_on_sc=True,
        needs_layout_passes=False,
        disable_bounds_checks=True,
        disable_semaphore_checks=True,
        skip_device_barrier=True,
    ),
)(_sc_body)


def kernel(x, y, A, B, c1, c2):
  x_arr = jnp.asarray(x, jnp.int32).reshape(1)
  y_arr = jnp.asarray(y, jnp.int32).reshape(1)
  cc = jnp.concatenate([c1, c2], axis=0).reshape(-1)
  return _sc_kernel(x_arr, y_arr, A.T, B.T, cc)


# speedup vs baseline: 1.3766x; 1.3766x over previous
import functools

import jax
import jax.numpy as jnp
from jax import lax
from jax.experimental import pallas as pl
from jax.experimental.pallas import tpu as pltpu
from jax.experimental.pallas import tpu_sc as plsc

DIM = 64
L = 16

_GATHER_DN = lax.GatherDimensionNumbers(
    offset_dims=(), collapsed_slice_dims=(0,), start_index_map=(0,))


def _shuffle(v, idx):
  return lax.gather(v, idx[:, None], _GATHER_DN, slice_sizes=(1,),
                    mode=lax.GatherScatterMode.PROMISE_IN_BOUNDS)


def _sc_body(x_hbm, y_hbm, at_hbm, bt_hbm, c1_hbm, c2_hbm, out_hbm,
             xi_v, col_a, col_b, cx_v, cy_v, out_v, sem):
  cid = lax.axis_index("c")
  sid = lax.axis_index("s")

  @pl.when(jnp.logical_and(cid == 0, sid == 0))
  def _():
    pltpu.sync_copy(x_hbm, xi_v.at[pl.ds(0, 1)])
    pltpu.sync_copy(y_hbm, xi_v.at[pl.ds(8, 1)])
    iv = xi_v[...]
    xs = iv[0]
    ys = iv[8]
    xt = pl.multiple_of((xs // 128) * 128, 128)
    yt = pl.multiple_of((ys // 128) * 128, 128)
    d0 = pltpu.make_async_copy(at_hbm.at[:, pl.ds(xt, 128)], col_a, sem)
    d1 = pltpu.make_async_copy(bt_hbm.at[:, pl.ds(yt, 128)], col_b, sem)
    d2 = pltpu.make_async_copy(c1_hbm.at[:, pl.ds(xt, 128)], cx_v, sem)
    d3 = pltpu.make_async_copy(c2_hbm.at[:, pl.ds(yt, 128)], cy_v, sem)
    d0.start()
    d1.start()
    d2.start()
    d3.start()
    d0.wait()
    d1.wait()
    d2.wait()
    d3.wait()

    ii = lax.iota(jnp.int32, L)
    xa = jnp.full((L,), xs - xt, jnp.int32)
    ya = jnp.full((L,), ys - yt, jnp.int32)
    acc = jnp.zeros((L,), jnp.float32)
    for i in range(DIM // L):
      av = plsc.load_gather(col_a, [ii + (i * L), xa])
      bv = plsc.load_gather(col_b, [ii + (i * L), ya])
      acc = acc + av * bv

    zz = jnp.zeros((L,), jnp.int32)
    cx = plsc.load_gather(cx_v, [zz, xa])
    cy = plsc.load_gather(cy_v, [zz, ya])
    zero = jnp.zeros((L,), jnp.float32)
    s = acc + jnp.where(ii == 0, cx + cy, zero)
    for k in (8, 4, 2, 1):
      s = s + _shuffle(s, ii ^ k)
    out_v[...] = s
    pltpu.sync_copy(out_v.at[pl.ds(0, 1)], out_hbm)


_sc_kernel = functools.partial(
    pl.kernel,
    out_type=jax.ShapeDtypeStruct((1,), jnp.float32),
    mesh=plsc.VectorSubcoreMesh(core_axis_name="c", subcore_axis_name="s",
                                num_cores=1),
    scratch_types=[
        pltpu.VMEM((L,), jnp.int32),
        pltpu.VMEM((DIM, 128), jnp.float32),
        pltpu.VMEM((DIM, 128), jnp.float32),
        pltpu.VMEM((1, 128), jnp.float32),
        pltpu.VMEM((1, 128), jnp.float32),
        pltpu.VMEM((L,), jnp.float32),
        pltpu.SemaphoreType.DMA,
    ],
    compiler_params=pltpu.CompilerParams(
        use_tc_tiling_on_sc=True,
        needs_layout_passes=False,
        disable_bounds_checks=True,
        disable_semaphore_checks=True,
        skip_device_barrier=True,
    ),
)(_sc_body)


def kernel(x, y, A, B, c1, c2):
  x_arr = jnp.asarray(x, jnp.int32).reshape(1)
  y_arr = jnp.asarray(y, jnp.int32).reshape(1)
  return _sc_kernel(x_arr, y_arr, A.T, B.T, c1.T, c2.T)


# speedup vs baseline: 1.4120x; 1.0257x over previous
import functools

import jax
import jax.numpy as jnp
from jax import lax
from jax.experimental import pallas as pl
from jax.experimental.pallas import tpu as pltpu
from jax.experimental.pallas import tpu_sc as plsc

DIM = 64
L = 16

_GATHER_DN = lax.GatherDimensionNumbers(
    offset_dims=(), collapsed_slice_dims=(0,), start_index_map=(0,))


def _shuffle(v, idx):
  return lax.gather(v, idx[:, None], _GATHER_DN, slice_sizes=(1,),
                    mode=lax.GatherScatterMode.PROMISE_IN_BOUNDS)


def _sc_body(x_hbm, y_hbm, at_hbm, bt_hbm, c1_hbm, c2_hbm, out_hbm,
             xi_v, col_a, col_b, cx_v, cy_v, out_v, sem):
  cid = lax.axis_index("c")
  sid = lax.axis_index("s")

  @pl.when(jnp.logical_and(cid == 0, sid == 0))
  def _():
    s0 = pltpu.make_async_copy(x_hbm, xi_v.at[pl.ds(0, 1)], sem)
    s1 = pltpu.make_async_copy(y_hbm, xi_v.at[pl.ds(8, 1)], sem)
    s0.start()
    s1.start()
    s0.wait()
    s1.wait()
    iv = xi_v[...]
    xs = iv[0]
    ys = iv[8]
    xt = pl.multiple_of((xs // 128) * 128, 128)
    yt = pl.multiple_of((ys // 128) * 128, 128)
    d0 = pltpu.make_async_copy(at_hbm.at[:, pl.ds(xt, 128)], col_a, sem)
    d1 = pltpu.make_async_copy(bt_hbm.at[:, pl.ds(yt, 128)], col_b, sem)
    d2 = pltpu.make_async_copy(c1_hbm.at[:, pl.ds(xt, 128)], cx_v, sem)
    d3 = pltpu.make_async_copy(c2_hbm.at[:, pl.ds(yt, 128)], cy_v, sem)
    d0.start()
    d1.start()
    d2.start()
    d3.start()
    d0.wait()
    d1.wait()
    d2.wait()
    d3.wait()

    ii = lax.iota(jnp.int32, L)
    xa = jnp.full((L,), xs - xt, jnp.int32)
    ya = jnp.full((L,), ys - yt, jnp.int32)
    acc = jnp.zeros((L,), jnp.float32)
    for i in range(DIM // L):
      av = plsc.load_gather(col_a, [ii + (i * L), xa])
      bv = plsc.load_gather(col_b, [ii + (i * L), ya])
      acc = acc + av * bv

    zz = jnp.zeros((L,), jnp.int32)
    cx = plsc.load_gather(cx_v, [zz, xa])
    cy = plsc.load_gather(cy_v, [zz, ya])
    zero = jnp.zeros((L,), jnp.float32)
    s = acc + jnp.where(ii == 0, cx + cy, zero)
    for k in (8, 4, 2, 1):
      s = s + _shuffle(s, ii ^ k)
    out_v[...] = s
    pltpu.sync_copy(out_v.at[pl.ds(0, 1)], out_hbm)


_sc_kernel = functools.partial(
    pl.kernel,
    out_type=jax.ShapeDtypeStruct((1,), jnp.float32),
    mesh=plsc.VectorSubcoreMesh(core_axis_name="c", subcore_axis_name="s",
                                num_cores=1, num_subcores=1),
    scratch_types=[
        pltpu.VMEM((L,), jnp.int32),
        pltpu.VMEM((DIM, 128), jnp.float32),
        pltpu.VMEM((DIM, 128), jnp.float32),
        pltpu.VMEM((1, 128), jnp.float32),
        pltpu.VMEM((1, 128), jnp.float32),
        pltpu.VMEM((L,), jnp.float32),
        pltpu.SemaphoreType.DMA,
    ],
    compiler_params=pltpu.CompilerParams(
        use_tc_tiling_on_sc=True,
        needs_layout_passes=False,
        disable_bounds_checks=True,
        disable_semaphore_checks=True,
        skip_device_barrier=True,
    ),
)(_sc_body)


def kernel(x, y, A, B, c1, c2):
  x_arr = jnp.asarray(x, jnp.int32).reshape(1)
  y_arr = jnp.asarray(y, jnp.int32).reshape(1)
  return _sc_kernel(x_arr, y_arr, A.T, B.T, c1.T, c2.T)


# speedup vs baseline: 1.4142x; 1.0015x over previous
import functools

import jax
import jax.numpy as jnp
from jax import lax
from jax.experimental import pallas as pl
from jax.experimental.pallas import tpu as pltpu
from jax.experimental.pallas import tpu_sc as plsc

DIM = 64
L = 16

_GATHER_DN = lax.GatherDimensionNumbers(
    offset_dims=(), collapsed_slice_dims=(0,), start_index_map=(0,))


def _shuffle(v, idx):
  return lax.gather(v, idx[:, None], _GATHER_DN, slice_sizes=(1,),
                    mode=lax.GatherScatterMode.PROMISE_IN_BOUNDS)


def _sc_body(xy_hbm, at_hbm, bt_hbm, c1_hbm, c2_hbm, out_hbm,
             xi_v, col_a, col_b, cx_v, cy_v, out_v, sem):
  cid = lax.axis_index("c")
  sid = lax.axis_index("s")

  @pl.when(jnp.logical_and(cid == 0, sid == 0))
  def _():
    pltpu.sync_copy(xy_hbm, xi_v.at[pl.ds(0, 2)])
    iv = xi_v[...]
    xs = iv[0]
    ys = iv[1]
    xt = pl.multiple_of((xs // 128) * 128, 128)
    yt = pl.multiple_of((ys // 128) * 128, 128)
    d0 = pltpu.make_async_copy(at_hbm.at[:, pl.ds(xt, 128)], col_a, sem)
    d1 = pltpu.make_async_copy(bt_hbm.at[:, pl.ds(yt, 128)], col_b, sem)
    d2 = pltpu.make_async_copy(c1_hbm.at[:, pl.ds(xt, 128)], cx_v, sem)
    d3 = pltpu.make_async_copy(c2_hbm.at[:, pl.ds(yt, 128)], cy_v, sem)
    d0.start()
    d1.start()
    d2.start()
    d3.start()
    d0.wait()
    d1.wait()
    d2.wait()
    d3.wait()

    ii = lax.iota(jnp.int32, L)
    xa = jnp.full((L,), xs - xt, jnp.int32)
    ya = jnp.full((L,), ys - yt, jnp.int32)
    acc = jnp.zeros((L,), jnp.float32)
    for i in range(DIM // L):
      av = plsc.load_gather(col_a, [ii + (i * L), xa])
      bv = plsc.load_gather(col_b, [ii + (i * L), ya])
      acc = acc + av * bv

    zz = jnp.zeros((L,), jnp.int32)
    cx = plsc.load_gather(cx_v, [zz, xa])
    cy = plsc.load_gather(cy_v, [zz, ya])
    zero = jnp.zeros((L,), jnp.float32)
    s = acc + jnp.where(ii == 0, cx + cy, zero)
    for k in (8, 4, 2, 1):
      s = s + _shuffle(s, ii ^ k)
    out_v[...] = s
    pltpu.sync_copy(out_v.at[pl.ds(0, 1)], out_hbm)


_sc_kernel = functools.partial(
    pl.kernel,
    out_type=jax.ShapeDtypeStruct((1,), jnp.float32),
    mesh=plsc.VectorSubcoreMesh(core_axis_name="c", subcore_axis_name="s",
                                num_cores=1, num_subcores=1),
    scratch_types=[
        pltpu.VMEM((L,), jnp.int32),
        pltpu.VMEM((DIM, 128), jnp.float32),
        pltpu.VMEM((DIM, 128), jnp.float32),
        pltpu.VMEM((1, 128), jnp.float32),
        pltpu.VMEM((1, 128), jnp.float32),
        pltpu.VMEM((L,), jnp.float32),
        pltpu.SemaphoreType.DMA,
    ],
    compiler_params=pltpu.CompilerParams(
        use_tc_tiling_on_sc=True,
        needs_layout_passes=False,
        disable_bounds_checks=True,
        disable_semaphore_checks=True,
        skip_device_barrier=True,
    ),
)(_sc_body)


def kernel(x, y, A, B, c1, c2):
  xy = jnp.stack([jnp.asarray(x, jnp.int32), jnp.asarray(y, jnp.int32)])
  return _sc_kernel(xy, A.T, B.T, c1.T, c2.T)
